# Initial kernel scaffold; baseline (speedup 1.0000x reference)
import jax
import jax.numpy as jnp
from jax.experimental import pallas as pl


def kernel(source_feat, target_feat, W_nt, b_nt, W_feat, b_feat, W_att, b_att, gat_bias, W_prep, b_prep, W_d0, b_d0, g_d0, be_d0, W_d1, b_d1, g_d1, be_d1, g_blk, be_blk, W_cls, b_cls, edge_index):
    raise NotImplementedError("write your pallas kernel here")


# speedup vs baseline: 8.2054x; 8.2054x over previous
import jax
import jax.numpy as jnp
from jax import lax
from jax.experimental import pallas as pl
from jax.experimental.pallas import tpu as pltpu
from jax.experimental.pallas import tpu_sc as plsc

N_NODES = 10000
NP = 10240
E = 320000
EP = 327680
D = 128
DH = 64
NS = 16
CHUNKS = 160
HCH = 80
CK = 128
STRIPE = NP // NS


def _pre_body(sf_ref, tf_ref, wnt_ref, bnt_ref, w1_ref, c1_ref, was_ref,
              cas_ref, wat_ref, cat_ref, head_ref, thl_ref, thr_ref,
              as_ref, at_ref):
    sf = sf_ref[...]
    tf = tf_ref[...]
    head_ref[...] = jnp.dot(sf, wnt_ref[...],
                            preferred_element_type=jnp.float32) + bnt_ref[...]
    th = jnp.dot(tf, w1_ref[...],
                 preferred_element_type=jnp.float32) + c1_ref[...]
    thl_ref[...] = th[:, :DH]
    thr_ref[...] = th[:, DH:]
    as_ref[...] = jnp.dot(sf, was_ref[...],
                          preferred_element_type=jnp.float32) + cas_ref[...]
    at_ref[...] = jnp.dot(tf, wat_ref[...],
                          preferred_element_type=jnp.float32) + cat_ref[...]


def _pre_pass(sfp, tfp, W_nt, b_nt2, W1, c1, Was, cas, Wat, cat):
    BR = 256
    grid = NP // BR
    row_spec = pl.BlockSpec((BR, D), lambda i: (i, 0))
    half_spec = pl.BlockSpec((BR, DH), lambda i: (i, 0))
    w_spec = pl.BlockSpec((D, D), lambda i: (0, 0))
    b_spec = pl.BlockSpec((1, D), lambda i: (0, 0))
    full = jax.ShapeDtypeStruct((NP, D), jnp.float32)
    half = jax.ShapeDtypeStruct((NP, DH), jnp.float32)
    return pl.pallas_call(
        _pre_body,
        grid=(grid,),
        in_specs=[row_spec, row_spec, w_spec, b_spec, w_spec, b_spec,
                  w_spec, b_spec, w_spec, b_spec],
        out_specs=[row_spec, half_spec, half_spec, row_spec, row_spec],
        out_shape=[full, half, half, full, full],
    )(sfp, tfp, W_nt, b_nt2, W1, c1, Was, cas, Wat, cat)


def _sc_body(src_hbm, tgt_hbm, asrc_hbm, atgt_hbm, thl_hbm, thr_hbm,
             ul_hbm, ur_hbm, d_hbm,
             src_v, tgt_v, as_v, at_v, ex_v, row_v, zrow_v, zd_v,
             u_sh, d_sh, sem):
    c = lax.axis_index("c")
    s = lax.axis_index("s")

    def _zero_zrow(i, _):
        for j in range(DH // 16):
            zrow_v[i, pl.ds(j * 16, 16)] = jnp.zeros((16,), jnp.float32)
        return 0
    lax.fori_loop(0, CK, _zero_zrow, 0)

    def _zero_zd(i, _):
        zd_v[pl.ds(i * 16, 16)] = jnp.zeros((16,), jnp.float32)
        return 0
    lax.fori_loop(0, STRIPE // 16, _zero_zd, 0)

    for k in range(STRIPE // CK):
        pltpu.sync_copy(zrow_v, u_sh.at[pl.ds(s * STRIPE + k * CK, CK)])
    pltpu.sync_copy(zd_v, d_sh.at[pl.ds(s * STRIPE, STRIPE)])
    plsc.subcore_barrier()

    pltpu.sync_copy(asrc_hbm, as_v)
    pltpu.sync_copy(atgt_hbm, at_v)

    for h in range(CHUNKS // HCH):
        pltpu.sync_copy(src_hbm.at[s * (CHUNKS // HCH) + h], src_v)
        pltpu.sync_copy(tgt_hbm.at[s * (CHUNKS // HCH) + h], tgt_v)

        def _edge_row(r, _):
            for j in range(CK // 16):
                sidx = src_v[r, pl.ds(j * 16, 16)]
                tidx = tgt_v[r, pl.ds(j * 16, 16)]
                a1 = plsc.load_gather(as_v, [sidx])
                a2 = plsc.load_gather(at_v, [tidx])
                z = a1 + a2
                w = jnp.exp(z * (-2.0))
                e = (1.0 - w) / (1.0 + w)
                ex_v[r, pl.ds(j * 16, 16)] = jnp.exp(e)
            return 0
        lax.fori_loop(0, HCH, _edge_row, 0)

        def _d_scatter(r, _):
            pltpu.sync_copy(ex_v.at[r], d_sh.at[src_v.at[r]], add=True)
            return 0
        lax.fori_loop(0, HCH, _d_scatter, 0)

        def _phase_b(th_ref):
            def _chunk(k, _):
                cp = pltpu.make_async_copy(th_ref.at[tgt_v.at[k]], row_v, sem)
                cp.start()
                cp.wait()

                def _scale(j, _2):
                    kk = jnp.full((16,), k, jnp.int32)
                    jj = jnp.full((16,), j, jnp.int32)
                    exs = plsc.load_gather(ex_v, [kk, jj])
                    for m in range(DH // 16):
                        row_v[j, pl.ds(m * 16, 16)] = (
                            row_v[j, pl.ds(m * 16, 16)] * exs)
                    return 0
                lax.fori_loop(0, CK, _scale, 0)
                pltpu.sync_copy(row_v, u_sh.at[src_v.at[k]], add=True)
                return 0
            lax.fori_loop(0, HCH, _chunk, 0)

        @pl.when(c == 0)
        def _():
            _phase_b(thl_hbm)

        @pl.when(c == 1)
        def _():
            _phase_b(thr_hbm)

    plsc.subcore_barrier()

    @pl.when(c == 0)
    def _():
        pltpu.sync_copy(u_sh.at[pl.ds(s * STRIPE, STRIPE)],
                        ul_hbm.at[pl.ds(s * STRIPE, STRIPE)])
        pltpu.sync_copy(d_sh.at[pl.ds(s * STRIPE, STRIPE)],
                        d_hbm.at[pl.ds(s * STRIPE, STRIPE)])

    @pl.when(c == 1)
    def _():
        pltpu.sync_copy(u_sh.at[pl.ds(s * STRIPE, STRIPE)],
                        ur_hbm.at[pl.ds(s * STRIPE, STRIPE)])


def _sc_aggregate(src3, tgt3, alpha_s, alpha_t, thl, thr):
    mesh = plsc.VectorSubcoreMesh(core_axis_name="c", subcore_axis_name="s")
    f32 = jnp.float32
    kern = pl.kernel(
        _sc_body,
        out_type=(jax.ShapeDtypeStruct((NP, DH), f32),
                  jax.ShapeDtypeStruct((NP, DH), f32),
                  jax.ShapeDtypeStruct((NP,), f32)),
        mesh=mesh,
        compiler_params=pltpu.CompilerParams(needs_layout_passes=False,
                                             use_tc_tiling_on_sc=False),
        scratch_types=[
            pltpu.VMEM((HCH, CK), jnp.int32),
            pltpu.VMEM((HCH, CK), jnp.int32),
            pltpu.VMEM((NP,), f32),
            pltpu.VMEM((NP,), f32),
            pltpu.VMEM((HCH, CK), f32),
            pltpu.VMEM((CK, DH), f32),
            pltpu.VMEM((CK, DH), f32),
            pltpu.VMEM((STRIPE,), f32),
            pltpu.VMEM_SHARED((NP, DH), f32),
            pltpu.VMEM_SHARED((NP,), f32),
            pltpu.SemaphoreType.DMA,
        ],
    )
    return kern(src3, tgt3, alpha_s, alpha_t, thl, thr)


def _ln(x, g, b, eps=1e-5):
    m = jnp.mean(x, axis=-1, keepdims=True)
    v = jnp.mean((x - m) * (x - m), axis=-1, keepdims=True)
    return (x - m) / jnp.sqrt(v + eps) * g + b


def _post_body(ul_ref, ur_ref, d3_ref, head_ref, gb_ref,
               wp_ref, bp_ref, w0_ref, b0_ref, g0_ref, be0_ref,
               w1_ref, b1_ref, g1_ref, be1_ref, gb2_ref, beb_ref,
               wc_ref, bc_ref, out_ref):
    u = jnp.concatenate([ul_ref[...], ur_ref[...]], axis=1)
    dsum = d3_ref[0, 0, :]
    recip = 1.0 / jnp.maximum(dsum, 1e-30)
    hp = u * recip[:, None] + gb_ref[...]
    h = (head_ref[...] + hp) * 0.5
    hs = jnp.dot(h, wp_ref[...], preferred_element_type=jnp.float32) + bp_ref[...]
    x = _ln(jnp.tanh(jnp.dot(hs, w0_ref[...],
                             preferred_element_type=jnp.float32) + b0_ref[...]),
            g0_ref[...], be0_ref[...])
    x = _ln(jnp.tanh(jnp.dot(x, w1_ref[...],
                             preferred_element_type=jnp.float32) + b1_ref[...]),
            g1_ref[...], be1_ref[...])
    x = _ln(jnp.tanh(hs + x), gb2_ref[...], beb_ref[...])
    out = jnp.dot(x, wc_ref[...], preferred_element_type=jnp.float32) + bc_ref[...]
    out_ref[...] = jax.nn.sigmoid(out)


def _post_pass(ul, ur, d3, head, gb, wp, bp, w0, b0, g0, be0,
               w1, b1, g1, be1, gblk, beblk, wc, bc):
    BR = 256
    grid = NP // BR
    row_spec = pl.BlockSpec((BR, D), lambda i: (i, 0))
    half_spec = pl.BlockSpec((BR, DH), lambda i: (i, 0))
    d_spec = pl.BlockSpec((1, 1, BR), lambda i: (i, 0, 0))
    w_spec = pl.BlockSpec((D, D), lambda i: (0, 0))
    b_spec = pl.BlockSpec((1, D), lambda i: (0, 0))
    return pl.pallas_call(
        _post_body,
        grid=(grid,),
        in_specs=[half_spec, half_spec, d_spec, row_spec, b_spec,
                  w_spec, b_spec, w_spec, b_spec, b_spec, b_spec,
                  w_spec, b_spec, b_spec, b_spec, b_spec, b_spec,
                  w_spec, b_spec],
        out_specs=row_spec,
        out_shape=jax.ShapeDtypeStruct((NP, D), jnp.float32),
    )(ul, ur, d3, head, gb, wp, bp, w0, b0, g0, be0,
      w1, b1, g1, be1, gblk, beblk, wc, bc)


def kernel(source_feat, target_feat, W_nt, b_nt, W_feat, b_feat, W_att, b_att,
           gat_bias, W_prep, b_prep, W_d0, b_d0, g_d0, be_d0, W_d1, b_d1,
           g_d1, be_d1, g_blk, be_blk, W_cls, b_cls, edge_index):
    f32 = jnp.float32
    W1 = W_nt @ W_feat
    c1 = b_nt @ W_feat + b_feat
    wa1 = W_att[:D, 0]
    wa2 = W_att[D:, 0]
    va_s = W1 @ wa1
    ca_s = jnp.dot(c1, wa1)
    va_t = W1 @ wa2
    ca_t = jnp.dot(c1, wa2) + b_att[0]
    Was = jnp.zeros((D, D), f32).at[:, 0].set(va_s)
    Wat = jnp.zeros((D, D), f32).at[:, 0].set(va_t)
    cas = jnp.zeros((1, D), f32).at[0, 0].set(ca_s)
    cat = jnp.zeros((1, D), f32).at[0, 0].set(ca_t)

    pad_n = ((0, NP - N_NODES), (0, 0))
    sfp = jnp.pad(source_feat, pad_n)
    tfp = jnp.pad(target_feat, pad_n)

    head, thl, thr, As, At = _pre_pass(
        sfp, tfp, W_nt, b_nt.reshape(1, D), W1, c1.reshape(1, D),
        Was, cas, Wat, cat)
    alpha_s = As[:, 0]
    alpha_t = At[:, 0]

    ei = edge_index.astype(jnp.int32)
    src3 = jnp.pad(ei[0], (0, EP - E), constant_values=N_NODES).reshape(
        NS * (CHUNKS // HCH), HCH, CK)
    tgt3 = jnp.pad(ei[1], (0, EP - E), constant_values=N_NODES).reshape(
        NS * (CHUNKS // HCH), HCH, CK)

    ul, ur, dd = _sc_aggregate(src3, tgt3, alpha_s, alpha_t, thl, thr)

    out = _post_pass(
        ul, ur, dd.reshape(NP // 256, 1, 256),
        head, gat_bias.reshape(1, D), W_prep, b_prep.reshape(1, D),
        W_d0, b_d0.reshape(1, D), g_d0.reshape(1, D), be_d0.reshape(1, D),
        W_d1, b_d1.reshape(1, D), g_d1.reshape(1, D), be_d1.reshape(1, D),
        g_blk.reshape(1, D), be_blk.reshape(1, D),
        jnp.zeros((D, D), f32).at[:, 0].set(W_cls[:, 0]),
        jnp.zeros((1, D), f32).at[0, 0].set(b_cls[0]))

    return out[:N_NODES, 0:1]


# speedup vs baseline: 10.2798x; 1.2528x over previous
import jax
import jax.numpy as jnp
from jax import lax
from jax.experimental import pallas as pl
from jax.experimental.pallas import tpu as pltpu
from jax.experimental.pallas import tpu_sc as plsc

N_NODES = 10000
NP = 10240
E = 320000
EP = 327680
D = 128
DH = 64
NS = 16
CHUNKS = 160
HCH = 80
CK = 128
STRIPE = NP // NS


def _pre_body(sf_ref, tf_ref, wnt_ref, bnt_ref, w1_ref, c1_ref, was_ref,
              cas_ref, wat_ref, cat_ref, head_ref, thl_ref, thr_ref,
              as_ref, at_ref):
    sf = sf_ref[...]
    tf = tf_ref[...]
    head_ref[...] = jnp.dot(sf, wnt_ref[...],
                            preferred_element_type=jnp.float32) + bnt_ref[...]
    th = jnp.dot(tf, w1_ref[...],
                 preferred_element_type=jnp.float32) + c1_ref[...]
    thl_ref[...] = th[:, :DH]
    thr_ref[...] = th[:, DH:]
    as_ref[...] = jnp.dot(sf, was_ref[...],
                          preferred_element_type=jnp.float32) + cas_ref[...]
    at_ref[...] = jnp.dot(tf, wat_ref[...],
                          preferred_element_type=jnp.float32) + cat_ref[...]


def _pre_pass(sfp, tfp, W_nt, b_nt2, W1, c1, Was, cas, Wat, cat):
    BR = 256
    grid = NP // BR
    row_spec = pl.BlockSpec((BR, D), lambda i: (i, 0))
    half_spec = pl.BlockSpec((BR, DH), lambda i: (i, 0))
    w_spec = pl.BlockSpec((D, D), lambda i: (0, 0))
    b_spec = pl.BlockSpec((1, D), lambda i: (0, 0))
    full = jax.ShapeDtypeStruct((NP, D), jnp.float32)
    half = jax.ShapeDtypeStruct((NP, DH), jnp.float32)
    return pl.pallas_call(
        _pre_body,
        grid=(grid,),
        in_specs=[row_spec, row_spec, w_spec, b_spec, w_spec, b_spec,
                  w_spec, b_spec, w_spec, b_spec],
        out_specs=[row_spec, half_spec, half_spec, row_spec, row_spec],
        out_shape=[full, half, half, full, full],
    )(sfp, tfp, W_nt, b_nt2, W1, c1, Was, cas, Wat, cat)


def _sc_body(src_hbm, tgt_hbm, asrc_hbm, atgt_hbm, thl_hbm, thr_hbm,
             ul_hbm, ur_hbm, d_hbm,
             src_v, tgt_v, as_v, at_v, ex_v, rowa_v, rowb_v, zrow_v, zd_v,
             u_sh, d_sh, sem_ga, sem_gb, sem_sa, sem_sb, semd):
    c = lax.axis_index("c")
    s = lax.axis_index("s")

    def _zero_zrow(i, _):
        for j in range(DH // 16):
            zrow_v[i, pl.ds(j * 16, 16)] = jnp.zeros((16,), jnp.float32)
        return 0
    lax.fori_loop(0, CK, _zero_zrow, 0)

    def _zero_zd(i, _):
        zd_v[pl.ds(i * 16, 16)] = jnp.zeros((16,), jnp.float32)
        return 0
    lax.fori_loop(0, STRIPE // 16, _zero_zd, 0)

    for k in range(STRIPE // CK):
        pltpu.sync_copy(zrow_v, u_sh.at[pl.ds(s * STRIPE + k * CK, CK)])
    pltpu.sync_copy(zd_v, d_sh.at[pl.ds(s * STRIPE, STRIPE)])
    plsc.subcore_barrier()

    pltpu.sync_copy(asrc_hbm, as_v)
    pltpu.sync_copy(atgt_hbm, at_v)

    for h in range(CHUNKS // HCH):
        pltpu.sync_copy(src_hbm.at[s * (CHUNKS // HCH) + h], src_v)
        pltpu.sync_copy(tgt_hbm.at[s * (CHUNKS // HCH) + h], tgt_v)

        def _edge_row(r, _):
            for j in range(CK // 16):
                sidx = src_v[r, pl.ds(j * 16, 16)]
                tidx = tgt_v[r, pl.ds(j * 16, 16)]
                a1 = plsc.load_gather(as_v, [sidx])
                a2 = plsc.load_gather(at_v, [tidx])
                z = a1 + a2
                w = jnp.exp(z * (-2.0))
                e = (1.0 - w) / (1.0 + w)
                ex_v[r, pl.ds(j * 16, 16)] = jnp.exp(e)
            pltpu.async_copy(ex_v.at[r], d_sh.at[src_v.at[r]], semd,
                             add=True)
            return 0
        lax.fori_loop(0, HCH, _edge_row, 0)

        def _scale(row_v, k):
            kk = jnp.full((16,), k, jnp.int32)

            def _scale4(q, _):
                for t in range(4):
                    j = q * 4 + t
                    jj = jnp.full((16,), j, jnp.int32)
                    exs = plsc.load_gather(ex_v, [kk, jj])
                    for m in range(DH // 16):
                        row_v[j, pl.ds(m * 16, 16)] = (
                            row_v[j, pl.ds(m * 16, 16)] * exs)
                return 0
            lax.fori_loop(0, CK // 4, _scale4, 0)

        def _phase_b(th_ref):
            pltpu.async_copy(th_ref.at[tgt_v.at[0]], rowa_v, sem_ga)
            pltpu.async_copy(th_ref.at[tgt_v.at[1]], rowb_v, sem_gb)
            P = HCH // 2

            def _pair(p, _):
                k0 = 2 * p
                k1 = k0 + 1
                pltpu.make_async_copy(th_ref.at[tgt_v.at[k0]], rowa_v,
                                      sem_ga).wait()
                _scale(rowa_v, k0)
                pltpu.async_copy(rowa_v, u_sh.at[src_v.at[k0]], sem_sa,
                                 add=True)
                pltpu.make_async_copy(th_ref.at[tgt_v.at[k1]], rowb_v,
                                      sem_gb).wait()
                _scale(rowb_v, k1)
                pltpu.async_copy(rowb_v, u_sh.at[src_v.at[k1]], sem_sb,
                                 add=True)

                @pl.when(p < P - 1)
                def _():
                    pltpu.make_async_copy(rowa_v, u_sh.at[src_v.at[k0]],
                                          sem_sa).wait()
                    pltpu.async_copy(th_ref.at[tgt_v.at[k0 + 2]], rowa_v,
                                     sem_ga)
                    pltpu.make_async_copy(rowb_v, u_sh.at[src_v.at[k1]],
                                          sem_sb).wait()
                    pltpu.async_copy(th_ref.at[tgt_v.at[k1 + 2]], rowb_v,
                                     sem_gb)
                return 0
            lax.fori_loop(0, P, _pair, 0)
            pltpu.make_async_copy(rowa_v, u_sh.at[src_v.at[HCH - 2]],
                                  sem_sa).wait()
            pltpu.make_async_copy(rowb_v, u_sh.at[src_v.at[HCH - 1]],
                                  sem_sb).wait()

        @pl.when(c == 0)
        def _():
            _phase_b(thl_hbm)

        @pl.when(c == 1)
        def _():
            _phase_b(thr_hbm)

        def _d_drain(r, _):
            pltpu.make_async_copy(ex_v.at[r], d_sh.at[src_v.at[r]],
                                  semd).wait()
            return 0
        lax.fori_loop(0, HCH, _d_drain, 0)

    plsc.subcore_barrier()

    @pl.when(c == 0)
    def _():
        pltpu.sync_copy(u_sh.at[pl.ds(s * STRIPE, STRIPE)],
                        ul_hbm.at[pl.ds(s * STRIPE, STRIPE)])
        pltpu.sync_copy(d_sh.at[pl.ds(s * STRIPE, STRIPE)],
                        d_hbm.at[pl.ds(s * STRIPE, STRIPE)])

    @pl.when(c == 1)
    def _():
        pltpu.sync_copy(u_sh.at[pl.ds(s * STRIPE, STRIPE)],
                        ur_hbm.at[pl.ds(s * STRIPE, STRIPE)])


def _sc_aggregate(src3, tgt3, alpha_s, alpha_t, thl, thr):
    mesh = plsc.VectorSubcoreMesh(core_axis_name="c", subcore_axis_name="s")
    f32 = jnp.float32
    kern = pl.kernel(
        _sc_body,
        out_type=(jax.ShapeDtypeStruct((NP, DH), f32),
                  jax.ShapeDtypeStruct((NP, DH), f32),
                  jax.ShapeDtypeStruct((NP,), f32)),
        mesh=mesh,
        compiler_params=pltpu.CompilerParams(needs_layout_passes=False,
                                             use_tc_tiling_on_sc=False),
        scratch_types=[
            pltpu.VMEM((HCH, CK), jnp.int32),
            pltpu.VMEM((HCH, CK), jnp.int32),
            pltpu.VMEM((NP,), f32),
            pltpu.VMEM((NP,), f32),
            pltpu.VMEM((HCH, CK), f32),
            pltpu.VMEM((CK, DH), f32),
            pltpu.VMEM((CK, DH), f32),
            pltpu.VMEM((CK, DH), f32),
            pltpu.VMEM((STRIPE,), f32),
            pltpu.VMEM_SHARED((NP, DH), f32),
            pltpu.VMEM_SHARED((NP,), f32),
            pltpu.SemaphoreType.DMA,
            pltpu.SemaphoreType.DMA,
            pltpu.SemaphoreType.DMA,
            pltpu.SemaphoreType.DMA,
            pltpu.SemaphoreType.DMA,
        ],
    )
    return kern(src3, tgt3, alpha_s, alpha_t, thl, thr)


def _ln(x, g, b, eps=1e-5):
    m = jnp.mean(x, axis=-1, keepdims=True)
    v = jnp.mean((x - m) * (x - m), axis=-1, keepdims=True)
    return (x - m) / jnp.sqrt(v + eps) * g + b


def _post_body(ul_ref, ur_ref, d3_ref, head_ref, gb_ref,
               wp_ref, bp_ref, w0_ref, b0_ref, g0_ref, be0_ref,
               w1_ref, b1_ref, g1_ref, be1_ref, gb2_ref, beb_ref,
               wc_ref, bc_ref, out_ref):
    u = jnp.concatenate([ul_ref[...], ur_ref[...]], axis=1)
    dsum = d3_ref[0, 0, :]
    recip = 1.0 / jnp.maximum(dsum, 1e-30)
    hp = u * recip[:, None] + gb_ref[...]
    h = (head_ref[...] + hp) * 0.5
    hs = jnp.dot(h, wp_ref[...], preferred_element_type=jnp.float32) + bp_ref[...]
    x = _ln(jnp.tanh(jnp.dot(hs, w0_ref[...],
                             preferred_element_type=jnp.float32) + b0_ref[...]),
            g0_ref[...], be0_ref[...])
    x = _ln(jnp.tanh(jnp.dot(x, w1_ref[...],
                             preferred_element_type=jnp.float32) + b1_ref[...]),
            g1_ref[...], be1_ref[...])
    x = _ln(jnp.tanh(hs + x), gb2_ref[...], beb_ref[...])
    out = jnp.dot(x, wc_ref[...], preferred_element_type=jnp.float32) + bc_ref[...]
    out_ref[...] = jax.nn.sigmoid(out)


def _post_pass(ul, ur, d3, head, gb, wp, bp, w0, b0, g0, be0,
               w1, b1, g1, be1, gblk, beblk, wc, bc):
    BR = 256
    grid = NP // BR
    row_spec = pl.BlockSpec((BR, D), lambda i: (i, 0))
    half_spec = pl.BlockSpec((BR, DH), lambda i: (i, 0))
    d_spec = pl.BlockSpec((1, 1, BR), lambda i: (i, 0, 0))
    w_spec = pl.BlockSpec((D, D), lambda i: (0, 0))
    b_spec = pl.BlockSpec((1, D), lambda i: (0, 0))
    return pl.pallas_call(
        _post_body,
        grid=(grid,),
        in_specs=[half_spec, half_spec, d_spec, row_spec, b_spec,
                  w_spec, b_spec, w_spec, b_spec, b_spec, b_spec,
                  w_spec, b_spec, b_spec, b_spec, b_spec, b_spec,
                  w_spec, b_spec],
        out_specs=row_spec,
        out_shape=jax.ShapeDtypeStruct((NP, D), jnp.float32),
    )(ul, ur, d3, head, gb, wp, bp, w0, b0, g0, be0,
      w1, b1, g1, be1, gblk, beblk, wc, bc)


def kernel(source_feat, target_feat, W_nt, b_nt, W_feat, b_feat, W_att, b_att,
           gat_bias, W_prep, b_prep, W_d0, b_d0, g_d0, be_d0, W_d1, b_d1,
           g_d1, be_d1, g_blk, be_blk, W_cls, b_cls, edge_index):
    f32 = jnp.float32
    W1 = W_nt @ W_feat
    c1 = b_nt @ W_feat + b_feat
    wa1 = W_att[:D, 0]
    wa2 = W_att[D:, 0]
    va_s = W1 @ wa1
    ca_s = jnp.dot(c1, wa1)
    va_t = W1 @ wa2
    ca_t = jnp.dot(c1, wa2) + b_att[0]
    Was = jnp.zeros((D, D), f32).at[:, 0].set(va_s)
    Wat = jnp.zeros((D, D), f32).at[:, 0].set(va_t)
    cas = jnp.zeros((1, D), f32).at[0, 0].set(ca_s)
    cat = jnp.zeros((1, D), f32).at[0, 0].set(ca_t)

    pad_n = ((0, NP - N_NODES), (0, 0))
    sfp = jnp.pad(source_feat, pad_n)
    tfp = jnp.pad(target_feat, pad_n)

    head, thl, thr, As, At = _pre_pass(
        sfp, tfp, W_nt, b_nt.reshape(1, D), W1, c1.reshape(1, D),
        Was, cas, Wat, cat)
    alpha_s = As[:, 0]
    alpha_t = At[:, 0]

    ei = edge_index.astype(jnp.int32)
    src3 = jnp.pad(ei[0], (0, EP - E), constant_values=N_NODES).reshape(
        NS * (CHUNKS // HCH), HCH, CK)
    tgt3 = jnp.pad(ei[1], (0, EP - E), constant_values=N_NODES).reshape(
        NS * (CHUNKS // HCH), HCH, CK)

    ul, ur, dd = _sc_aggregate(src3, tgt3, alpha_s, alpha_t, thl, thr)

    out = _post_pass(
        ul, ur, dd.reshape(NP // 256, 1, 256),
        head, gat_bias.reshape(1, D), W_prep, b_prep.reshape(1, D),
        W_d0, b_d0.reshape(1, D), g_d0.reshape(1, D), be_d0.reshape(1, D),
        W_d1, b_d1.reshape(1, D), g_d1.reshape(1, D), be_d1.reshape(1, D),
        g_blk.reshape(1, D), be_blk.reshape(1, D),
        jnp.zeros((D, D), f32).at[:, 0].set(W_cls[:, 0]),
        jnp.zeros((1, D), f32).at[0, 0].set(b_cls[0]))

    return out[:N_NODES, 0:1]


# speedup vs baseline: 11.2953x; 1.0988x over previous
import jax
import jax.numpy as jnp
from jax import lax
from jax.experimental import pallas as pl
from jax.experimental.pallas import tpu as pltpu
from jax.experimental.pallas import tpu_sc as plsc

N_NODES = 10000
NP = 10240
E = 320000
EP = 327680
D = 128
DH = 64
NS = 16
CHUNKS = 160
HCH = 80
CK = 128
STRIPE = NP // NS


def _pre_body(sf_ref, tf_ref, wnt_ref, bnt_ref, w1_ref, c1_ref, was_ref,
              cas_ref, wat_ref, cat_ref, head_ref, thl_ref, thr_ref,
              as_ref, at_ref):
    sf = sf_ref[...]
    tf = tf_ref[...]
    head_ref[...] = jnp.dot(sf, wnt_ref[...],
                            preferred_element_type=jnp.float32) + bnt_ref[...]
    th = jnp.dot(tf, w1_ref[...],
                 preferred_element_type=jnp.float32) + c1_ref[...]
    thl_ref[...] = th[:, :DH]
    thr_ref[...] = th[:, DH:]
    as_ref[...] = jnp.dot(sf, was_ref[...],
                          preferred_element_type=jnp.float32) + cas_ref[...]
    at_ref[...] = jnp.dot(tf, wat_ref[...],
                          preferred_element_type=jnp.float32) + cat_ref[...]


def _pre_pass(sfp, tfp, W_nt, b_nt2, W1, c1, Was, cas, Wat, cat):
    BR = 256
    grid = NP // BR
    row_spec = pl.BlockSpec((BR, D), lambda i: (i, 0))
    half_spec = pl.BlockSpec((BR, DH), lambda i: (i, 0))
    w_spec = pl.BlockSpec((D, D), lambda i: (0, 0))
    b_spec = pl.BlockSpec((1, D), lambda i: (0, 0))
    full = jax.ShapeDtypeStruct((NP, D), jnp.float32)
    half = jax.ShapeDtypeStruct((NP, DH), jnp.float32)
    return pl.pallas_call(
        _pre_body,
        grid=(grid,),
        in_specs=[row_spec, row_spec, w_spec, b_spec, w_spec, b_spec,
                  w_spec, b_spec, w_spec, b_spec],
        out_specs=[row_spec, half_spec, half_spec, row_spec, row_spec],
        out_shape=[full, half, half, full, full],
    )(sfp, tfp, W_nt, b_nt2, W1, c1, Was, cas, Wat, cat)


def _sc_body(src_hbm, tgt_hbm, asrc_hbm, atgt_hbm, thl_hbm, thr_hbm,
             ul_hbm, ur_hbm, d_hbm,
             src_v, tgt_v, as_v, at_v, ex_v, row0_v, row1_v, row2_v, row3_v,
             zd_v, u_sh, d_sh, sg0, sg1, sg2, sg3, ss0, ss1, ss2, ss3, semd):
    c = lax.axis_index("c")
    s = lax.axis_index("s")
    row_vs = [row0_v, row1_v, row2_v, row3_v]
    sem_gs = [sg0, sg1, sg2, sg3]
    sem_ss = [ss0, ss1, ss2, ss3]
    NB = len(row_vs)

    def _zero_zrow(i, _):
        for j in range(DH // 16):
            row_vs[0][i, pl.ds(j * 16, 16)] = jnp.zeros((16,), jnp.float32)
        return 0
    lax.fori_loop(0, CK, _zero_zrow, 0)

    def _zero_zd(i, _):
        zd_v[pl.ds(i * 16, 16)] = jnp.zeros((16,), jnp.float32)
        return 0
    lax.fori_loop(0, STRIPE // 16, _zero_zd, 0)

    for k in range(STRIPE // CK):
        pltpu.sync_copy(row_vs[0], u_sh.at[pl.ds(s * STRIPE + k * CK, CK)])
    pltpu.sync_copy(zd_v, d_sh.at[pl.ds(s * STRIPE, STRIPE)])
    plsc.subcore_barrier()

    pltpu.sync_copy(asrc_hbm, as_v)
    pltpu.sync_copy(atgt_hbm, at_v)

    def _ex_row(k):
        for j in range(CK // 16):
            sidx = src_v[k, pl.ds(j * 16, 16)]
            tidx = tgt_v[k, pl.ds(j * 16, 16)]
            a1 = plsc.load_gather(as_v, [sidx])
            a2 = plsc.load_gather(at_v, [tidx])
            z = a1 + a2
            w = jnp.exp(z * (-2.0))
            e = (1.0 - w) / (1.0 + w)
            ex_v[k, pl.ds(j * 16, 16)] = jnp.exp(e)

    def _scale(row_v, k):
        kk = jnp.full((16,), k, jnp.int32)

        def _scale4(q, _):
            for t in range(4):
                j = q * 4 + t
                jj = jnp.full((16,), j, jnp.int32)
                exs = plsc.load_gather(ex_v, [kk, jj])
                for m in range(DH // 16):
                    row_v[j, pl.ds(m * 16, 16)] = (
                        row_v[j, pl.ds(m * 16, 16)] * exs)
            return 0
        lax.fori_loop(0, CK // 4, _scale4, 0)

    for h in range(CHUNKS // HCH):
        pltpu.sync_copy(src_hbm.at[s * (CHUNKS // HCH) + h], src_v)
        pltpu.sync_copy(tgt_hbm.at[s * (CHUNKS // HCH) + h], tgt_v)

        def _run_half(th_ref):
            for b in range(NB):
                pltpu.async_copy(th_ref.at[tgt_v.at[b]], row_vs[b],
                                 sem_gs[b])

            def _group(p, _):
                for b in range(NB):
                    k = NB * p + b
                    _ex_row(k)

                    @pl.when(c == 0)
                    def _():
                        pltpu.async_copy(ex_v.at[k], d_sh.at[src_v.at[k]],
                                         semd, add=True)
                    pltpu.make_async_copy(th_ref.at[tgt_v.at[k]],
                                          row_vs[b], sem_gs[b]).wait()
                    _scale(row_vs[b], k)
                    pltpu.async_copy(row_vs[b], u_sh.at[src_v.at[k]],
                                     sem_ss[b], add=True)

                    @pl.when(k < HCH - NB)
                    def _():
                        pltpu.make_async_copy(row_vs[b],
                                              u_sh.at[src_v.at[k]],
                                              sem_ss[b]).wait()
                        pltpu.async_copy(th_ref.at[tgt_v.at[k + NB]],
                                         row_vs[b], sem_gs[b])
                return 0
            lax.fori_loop(0, HCH // NB, _group, 0)
            for b in range(NB):
                pltpu.make_async_copy(row_vs[b],
                                      u_sh.at[src_v.at[HCH - NB + b]],
                                      sem_ss[b]).wait()

        @pl.when(c == 0)
        def _():
            _run_half(thl_hbm)

        @pl.when(c == 1)
        def _():
            _run_half(thr_hbm)

        @pl.when(c == 0)
        def _():
            def _d_drain(r, _):
                pltpu.make_async_copy(ex_v.at[r], d_sh.at[src_v.at[r]],
                                      semd).wait()
                return 0
            lax.fori_loop(0, HCH, _d_drain, 0)

    plsc.subcore_barrier()

    @pl.when(c == 0)
    def _():
        pltpu.sync_copy(u_sh.at[pl.ds(s * STRIPE, STRIPE)],
                        ul_hbm.at[pl.ds(s * STRIPE, STRIPE)])
        pltpu.sync_copy(d_sh.at[pl.ds(s * STRIPE, STRIPE)],
                        d_hbm.at[pl.ds(s * STRIPE, STRIPE)])

    @pl.when(c == 1)
    def _():
        pltpu.sync_copy(u_sh.at[pl.ds(s * STRIPE, STRIPE)],
                        ur_hbm.at[pl.ds(s * STRIPE, STRIPE)])


def _sc_aggregate(src3, tgt3, alpha_s, alpha_t, thl, thr):
    mesh = plsc.VectorSubcoreMesh(core_axis_name="c", subcore_axis_name="s")
    f32 = jnp.float32
    kern = pl.kernel(
        _sc_body,
        out_type=(jax.ShapeDtypeStruct((NP, DH), f32),
                  jax.ShapeDtypeStruct((NP, DH), f32),
                  jax.ShapeDtypeStruct((NP,), f32)),
        mesh=mesh,
        compiler_params=pltpu.CompilerParams(needs_layout_passes=False,
                                             use_tc_tiling_on_sc=False),
        scratch_types=[
            pltpu.VMEM((HCH, CK), jnp.int32),
            pltpu.VMEM((HCH, CK), jnp.int32),
            pltpu.VMEM((NP,), f32),
            pltpu.VMEM((NP,), f32),
            pltpu.VMEM((HCH, CK), f32),
            pltpu.VMEM((CK, DH), f32),
            pltpu.VMEM((CK, DH), f32),
            pltpu.VMEM((CK, DH), f32),
            pltpu.VMEM((CK, DH), f32),
            pltpu.VMEM((STRIPE,), f32),
            pltpu.VMEM_SHARED((NP, DH), f32),
            pltpu.VMEM_SHARED((NP,), f32),
            pltpu.SemaphoreType.DMA,
            pltpu.SemaphoreType.DMA,
            pltpu.SemaphoreType.DMA,
            pltpu.SemaphoreType.DMA,
            pltpu.SemaphoreType.DMA,
            pltpu.SemaphoreType.DMA,
            pltpu.SemaphoreType.DMA,
            pltpu.SemaphoreType.DMA,
            pltpu.SemaphoreType.DMA,
        ],
    )
    return kern(src3, tgt3, alpha_s, alpha_t, thl, thr)


def _ln(x, g, b, eps=1e-5):
    m = jnp.mean(x, axis=-1, keepdims=True)
    v = jnp.mean((x - m) * (x - m), axis=-1, keepdims=True)
    return (x - m) / jnp.sqrt(v + eps) * g + b


def _post_body(ul_ref, ur_ref, d3_ref, head_ref, gb_ref,
               wp_ref, bp_ref, w0_ref, b0_ref, g0_ref, be0_ref,
               w1_ref, b1_ref, g1_ref, be1_ref, gb2_ref, beb_ref,
               wc_ref, bc_ref, out_ref):
    u = jnp.concatenate([ul_ref[...], ur_ref[...]], axis=1)
    dsum = d3_ref[0, 0, :]
    recip = 1.0 / jnp.maximum(dsum, 1e-30)
    hp = u * recip[:, None] + gb_ref[...]
    h = (head_ref[...] + hp) * 0.5
    hs = jnp.dot(h, wp_ref[...], preferred_element_type=jnp.float32) + bp_ref[...]
    x = _ln(jnp.tanh(jnp.dot(hs, w0_ref[...],
                             preferred_element_type=jnp.float32) + b0_ref[...]),
            g0_ref[...], be0_ref[...])
    x = _ln(jnp.tanh(jnp.dot(x, w1_ref[...],
                             preferred_element_type=jnp.float32) + b1_ref[...]),
            g1_ref[...], be1_ref[...])
    x = _ln(jnp.tanh(hs + x), gb2_ref[...], beb_ref[...])
    out = jnp.dot(x, wc_ref[...], preferred_element_type=jnp.float32) + bc_ref[...]
    out_ref[...] = jax.nn.sigmoid(out)


def _post_pass(ul, ur, d3, head, gb, wp, bp, w0, b0, g0, be0,
               w1, b1, g1, be1, gblk, beblk, wc, bc):
    BR = 256
    grid = NP // BR
    row_spec = pl.BlockSpec((BR, D), lambda i: (i, 0))
    half_spec = pl.BlockSpec((BR, DH), lambda i: (i, 0))
    d_spec = pl.BlockSpec((1, 1, BR), lambda i: (i, 0, 0))
    w_spec = pl.BlockSpec((D, D), lambda i: (0, 0))
    b_spec = pl.BlockSpec((1, D), lambda i: (0, 0))
    return pl.pallas_call(
        _post_body,
        grid=(grid,),
        in_specs=[half_spec, half_spec, d_spec, row_spec, b_spec,
                  w_spec, b_spec, w_spec, b_spec, b_spec, b_spec,
                  w_spec, b_spec, b_spec, b_spec, b_spec, b_spec,
                  w_spec, b_spec],
        out_specs=row_spec,
        out_shape=jax.ShapeDtypeStruct((NP, D), jnp.float32),
    )(ul, ur, d3, head, gb, wp, bp, w0, b0, g0, be0,
      w1, b1, g1, be1, gblk, beblk, wc, bc)


def kernel(source_feat, target_feat, W_nt, b_nt, W_feat, b_feat, W_att, b_att,
           gat_bias, W_prep, b_prep, W_d0, b_d0, g_d0, be_d0, W_d1, b_d1,
           g_d1, be_d1, g_blk, be_blk, W_cls, b_cls, edge_index):
    f32 = jnp.float32
    W1 = W_nt @ W_feat
    c1 = b_nt @ W_feat + b_feat
    wa1 = W_att[:D, 0]
    wa2 = W_att[D:, 0]
    va_s = W1 @ wa1
    ca_s = jnp.dot(c1, wa1)
    va_t = W1 @ wa2
    ca_t = jnp.dot(c1, wa2) + b_att[0]
    Was = jnp.zeros((D, D), f32).at[:, 0].set(va_s)
    Wat = jnp.zeros((D, D), f32).at[:, 0].set(va_t)
    cas = jnp.zeros((1, D), f32).at[0, 0].set(ca_s)
    cat = jnp.zeros((1, D), f32).at[0, 0].set(ca_t)

    pad_n = ((0, NP - N_NODES), (0, 0))
    sfp = jnp.pad(source_feat, pad_n)
    tfp = jnp.pad(target_feat, pad_n)

    head, thl, thr, As, At = _pre_pass(
        sfp, tfp, W_nt, b_nt.reshape(1, D), W1, c1.reshape(1, D),
        Was, cas, Wat, cat)
    alpha_s = As[:, 0]
    alpha_t = At[:, 0]

    ei = edge_index.astype(jnp.int32)
    src3 = jnp.pad(ei[0], (0, EP - E), constant_values=N_NODES).reshape(
        NS * (CHUNKS // HCH), HCH, CK)
    tgt3 = jnp.pad(ei[1], (0, EP - E), constant_values=N_NODES).reshape(
        NS * (CHUNKS // HCH), HCH, CK)

    ul, ur, dd = _sc_aggregate(src3, tgt3, alpha_s, alpha_t, thl, thr)

    out = _post_pass(
        ul, ur, dd.reshape(NP // 256, 1, 256),
        head, gat_bias.reshape(1, D), W_prep, b_prep.reshape(1, D),
        W_d0, b_d0.reshape(1, D), g_d0.reshape(1, D), be_d0.reshape(1, D),
        W_d1, b_d1.reshape(1, D), g_d1.reshape(1, D), be_d1.reshape(1, D),
        g_blk.reshape(1, D), be_blk.reshape(1, D),
        jnp.zeros((D, D), f32).at[:, 0].set(W_cls[:, 0]),
        jnp.zeros((1, D), f32).at[0, 0].set(b_cls[0]))

    return out[:N_NODES, 0:1]


# speedup vs baseline: 13.2916x; 1.1767x over previous
import jax
import jax.numpy as jnp
from jax import lax
from jax.experimental import pallas as pl
from jax.experimental.pallas import tpu as pltpu
from jax.experimental.pallas import tpu_sc as plsc

N_NODES = 10000
NP = 10240
E = 320000
EP = 327680
D = 128
DH = 64
NS = 16
CHUNKS = 160
HCH = 80
CK = 128
STRIPE = NP // NS


def _pre_body(sf_ref, tf_ref, wnt_ref, bnt_ref, w1_ref, c1_ref, was_ref,
              cas_ref, wat_ref, cat_ref, head_ref, thl_ref, thr_ref,
              as_ref, at_ref):
    sf = sf_ref[...]
    tf = tf_ref[...]
    head_ref[...] = jnp.dot(sf, wnt_ref[...],
                            preferred_element_type=jnp.float32) + bnt_ref[...]
    th = jnp.dot(tf, w1_ref[...],
                 preferred_element_type=jnp.float32) + c1_ref[...]
    thl_ref[...] = th[:, :DH]
    thr_ref[...] = th[:, DH:]
    as_ref[...] = jnp.dot(sf, was_ref[...],
                          preferred_element_type=jnp.float32) + cas_ref[...]
    at_ref[...] = jnp.dot(tf, wat_ref[...],
                          preferred_element_type=jnp.float32) + cat_ref[...]


def _pre_pass(sfp, tfp, W_nt, b_nt2, W1, c1, Was, cas, Wat, cat):
    BR = 256
    grid = NP // BR
    row_spec = pl.BlockSpec((BR, D), lambda i: (i, 0))
    half_spec = pl.BlockSpec((BR, DH), lambda i: (i, 0))
    w_spec = pl.BlockSpec((D, D), lambda i: (0, 0))
    b_spec = pl.BlockSpec((1, D), lambda i: (0, 0))
    full = jax.ShapeDtypeStruct((NP, D), jnp.float32)
    half = jax.ShapeDtypeStruct((NP, DH), jnp.float32)
    return pl.pallas_call(
        _pre_body,
        grid=(grid,),
        in_specs=[row_spec, row_spec, w_spec, b_spec, w_spec, b_spec,
                  w_spec, b_spec, w_spec, b_spec],
        out_specs=[row_spec, half_spec, half_spec, row_spec, row_spec],
        out_shape=[full, half, half, full, full],
    )(sfp, tfp, W_nt, b_nt2, W1, c1, Was, cas, Wat, cat)


def _sc_body(src_hbm, tgt_hbm, asrc_hbm, atgt_hbm, thl_hbm, thr_hbm,
             ul_hbm, ur_hbm, d_hbm,
             src_v, tgt_v, as_v, at_v, ex_v, in0_v, in1_v, in2_v, in3_v,
             out0_v, out1_v, zd_v, u_sh, d_sh,
             sg0, sg1, sg2, sg3, ss0, ss1, semd):
    c = lax.axis_index("c")
    s = lax.axis_index("s")
    in_vs = [in0_v, in1_v, in2_v, in3_v]
    out_vs = [out0_v, out1_v]
    sem_gs = [sg0, sg1, sg2, sg3]
    sem_ss = [ss0, ss1]
    NB = len(in_vs)

    def _zero_zrow(i, _):
        for j in range(DH // 16):
            out0_v[i, pl.ds(j * 16, 16)] = jnp.zeros((16,), jnp.float32)
        return 0
    lax.fori_loop(0, CK, _zero_zrow, 0)

    def _zero_zd(i, _):
        zd_v[pl.ds(i * 16, 16)] = jnp.zeros((16,), jnp.float32)
        return 0
    lax.fori_loop(0, STRIPE // 16, _zero_zd, 0)

    for k in range(STRIPE // CK):
        pltpu.sync_copy(out0_v, u_sh.at[pl.ds(s * STRIPE + k * CK, CK)])
    pltpu.sync_copy(zd_v, d_sh.at[pl.ds(s * STRIPE, STRIPE)])
    plsc.subcore_barrier()

    pltpu.sync_copy(asrc_hbm, as_v)
    pltpu.sync_copy(atgt_hbm, at_v)

    def _ex_row(k):
        for j in range(CK // 16):
            sidx = src_v[k, pl.ds(j * 16, 16)]
            tidx = tgt_v[k, pl.ds(j * 16, 16)]
            a1 = plsc.load_gather(as_v, [sidx])
            a2 = plsc.load_gather(at_v, [tidx])
            z = a1 + a2
            w = jnp.exp(z * (-2.0))
            e = (1.0 - w) / (1.0 + w)
            ex_v[k, pl.ds(j * 16, 16)] = jnp.exp(e)

    def _scale(in_v, out_v, k):
        kk = jnp.full((16,), k, jnp.int32)

        def _scale4(q, _):
            for t in range(4):
                j = q * 4 + t
                jj = jnp.full((16,), j, jnp.int32)
                exs = plsc.load_gather(ex_v, [kk, jj])
                for m in range(DH // 32):
                    pw = in_v[j, pl.ds(m * 16, 16)]
                    bb = plsc.bitcast(pw, jnp.bfloat16)
                    lo, hi = plsc.unpack(
                        bb, format=plsc.PackFormat.INTERLEAVED)
                    out_v[j, pl.ds(m * 32, 16)] = lo * exs
                    out_v[j, pl.ds(m * 32 + 16, 16)] = hi * exs
            return 0
        lax.fori_loop(0, CK // 4, _scale4, 0)

    for h in range(CHUNKS // HCH):
        pltpu.sync_copy(src_hbm.at[s * (CHUNKS // HCH) + h], src_v)
        pltpu.sync_copy(tgt_hbm.at[s * (CHUNKS // HCH) + h], tgt_v)

        def _run_half(th_ref):
            for b in range(NB):
                pltpu.async_copy(th_ref.at[tgt_v.at[b]], in_vs[b],
                                 sem_gs[b])

            def _group(p, _):
                for b in range(NB):
                    k = NB * p + b
                    o = b % 2
                    _ex_row(k)

                    @pl.when(c == 0)
                    def _():
                        pltpu.async_copy(ex_v.at[k], d_sh.at[src_v.at[k]],
                                         semd, add=True)
                    pltpu.make_async_copy(th_ref.at[tgt_v.at[k]],
                                          in_vs[b], sem_gs[b]).wait()

                    @pl.when(k >= 2)
                    def _():
                        pltpu.make_async_copy(out_vs[o],
                                              u_sh.at[src_v.at[k - 2]],
                                              sem_ss[o]).wait()
                    _scale(in_vs[b], out_vs[o], k)
                    pltpu.async_copy(out_vs[o], u_sh.at[src_v.at[k]],
                                     sem_ss[o], add=True)

                    @pl.when(k < HCH - NB)
                    def _():
                        pltpu.async_copy(th_ref.at[tgt_v.at[k + NB]],
                                         in_vs[b], sem_gs[b])
                return 0
            lax.fori_loop(0, HCH // NB, _group, 0)
            for o in range(2):
                pltpu.make_async_copy(out_vs[o],
                                      u_sh.at[src_v.at[HCH - 2 + o]],
                                      sem_ss[o]).wait()

        @pl.when(c == 0)
        def _():
            _run_half(thl_hbm)

        @pl.when(c == 1)
        def _():
            _run_half(thr_hbm)

        @pl.when(c == 0)
        def _():
            def _d_drain(r, _):
                pltpu.make_async_copy(ex_v.at[r], d_sh.at[src_v.at[r]],
                                      semd).wait()
                return 0
            lax.fori_loop(0, HCH, _d_drain, 0)

    plsc.subcore_barrier()

    @pl.when(c == 0)
    def _():
        pltpu.sync_copy(u_sh.at[pl.ds(s * STRIPE, STRIPE)],
                        ul_hbm.at[pl.ds(s * STRIPE, STRIPE)])
        pltpu.sync_copy(d_sh.at[pl.ds(s * STRIPE, STRIPE)],
                        d_hbm.at[pl.ds(s * STRIPE, STRIPE)])

    @pl.when(c == 1)
    def _():
        pltpu.sync_copy(u_sh.at[pl.ds(s * STRIPE, STRIPE)],
                        ur_hbm.at[pl.ds(s * STRIPE, STRIPE)])


def _sc_aggregate(src3, tgt3, alpha_s, alpha_t, thl, thr):
    mesh = plsc.VectorSubcoreMesh(core_axis_name="c", subcore_axis_name="s")
    f32 = jnp.float32
    kern = pl.kernel(
        _sc_body,
        out_type=(jax.ShapeDtypeStruct((NP, DH), f32),
                  jax.ShapeDtypeStruct((NP, DH), f32),
                  jax.ShapeDtypeStruct((NP,), f32)),
        mesh=mesh,
        compiler_params=pltpu.CompilerParams(needs_layout_passes=False,
                                             use_tc_tiling_on_sc=False),
        scratch_types=[
            pltpu.VMEM((HCH, CK), jnp.int32),
            pltpu.VMEM((HCH, CK), jnp.int32),
            pltpu.VMEM((NP,), f32),
            pltpu.VMEM((NP,), f32),
            pltpu.VMEM((HCH, CK), f32),
            pltpu.VMEM((CK, DH // 2), f32),
            pltpu.VMEM((CK, DH // 2), f32),
            pltpu.VMEM((CK, DH // 2), f32),
            pltpu.VMEM((CK, DH // 2), f32),
            pltpu.VMEM((CK, DH), f32),
            pltpu.VMEM((CK, DH), f32),
            pltpu.VMEM((STRIPE,), f32),
            pltpu.VMEM_SHARED((NP, DH), f32),
            pltpu.VMEM_SHARED((NP,), f32),
            pltpu.SemaphoreType.DMA,
            pltpu.SemaphoreType.DMA,
            pltpu.SemaphoreType.DMA,
            pltpu.SemaphoreType.DMA,
            pltpu.SemaphoreType.DMA,
            pltpu.SemaphoreType.DMA,
            pltpu.SemaphoreType.DMA,
        ],
    )
    return kern(src3, tgt3, alpha_s, alpha_t, thl, thr)


def _ln(x, g, b, eps=1e-5):
    m = jnp.mean(x, axis=-1, keepdims=True)
    v = jnp.mean((x - m) * (x - m), axis=-1, keepdims=True)
    return (x - m) / jnp.sqrt(v + eps) * g + b


def _post_body(ul_ref, ur_ref, d3_ref, head_ref, gb_ref,
               wp_ref, bp_ref, w0_ref, b0_ref, g0_ref, be0_ref,
               w1_ref, b1_ref, g1_ref, be1_ref, gb2_ref, beb_ref,
               wc_ref, bc_ref, out_ref):
    u = jnp.concatenate([ul_ref[...], ur_ref[...]], axis=1)
    dsum = d3_ref[0, 0, :]
    recip = 1.0 / jnp.maximum(dsum, 1e-30)
    hp = u * recip[:, None] + gb_ref[...]
    h = (head_ref[...] + hp) * 0.5
    hs = jnp.dot(h, wp_ref[...], preferred_element_type=jnp.float32) + bp_ref[...]
    x = _ln(jnp.tanh(jnp.dot(hs, w0_ref[...],
                             preferred_element_type=jnp.float32) + b0_ref[...]),
            g0_ref[...], be0_ref[...])
    x = _ln(jnp.tanh(jnp.dot(x, w1_ref[...],
                             preferred_element_type=jnp.float32) + b1_ref[...]),
            g1_ref[...], be1_ref[...])
    x = _ln(jnp.tanh(hs + x), gb2_ref[...], beb_ref[...])
    out = jnp.dot(x, wc_ref[...], preferred_element_type=jnp.float32) + bc_ref[...]
    out_ref[...] = jax.nn.sigmoid(out)


def _post_pass(ul, ur, d3, head, gb, wp, bp, w0, b0, g0, be0,
               w1, b1, g1, be1, gblk, beblk, wc, bc):
    BR = 256
    grid = NP // BR
    row_spec = pl.BlockSpec((BR, D), lambda i: (i, 0))
    half_spec = pl.BlockSpec((BR, DH), lambda i: (i, 0))
    d_spec = pl.BlockSpec((1, 1, BR), lambda i: (i, 0, 0))
    w_spec = pl.BlockSpec((D, D), lambda i: (0, 0))
    b_spec = pl.BlockSpec((1, D), lambda i: (0, 0))
    return pl.pallas_call(
        _post_body,
        grid=(grid,),
        in_specs=[half_spec, half_spec, d_spec, row_spec, b_spec,
                  w_spec, b_spec, w_spec, b_spec, b_spec, b_spec,
                  w_spec, b_spec, b_spec, b_spec, b_spec, b_spec,
                  w_spec, b_spec],
        out_specs=row_spec,
        out_shape=jax.ShapeDtypeStruct((NP, D), jnp.float32),
    )(ul, ur, d3, head, gb, wp, bp, w0, b0, g0, be0,
      w1, b1, g1, be1, gblk, beblk, wc, bc)


def kernel(source_feat, target_feat, W_nt, b_nt, W_feat, b_feat, W_att, b_att,
           gat_bias, W_prep, b_prep, W_d0, b_d0, g_d0, be_d0, W_d1, b_d1,
           g_d1, be_d1, g_blk, be_blk, W_cls, b_cls, edge_index):
    f32 = jnp.float32
    W1 = W_nt @ W_feat
    c1 = b_nt @ W_feat + b_feat
    wa1 = W_att[:D, 0]
    wa2 = W_att[D:, 0]
    va_s = W1 @ wa1
    ca_s = jnp.dot(c1, wa1)
    va_t = W1 @ wa2
    ca_t = jnp.dot(c1, wa2) + b_att[0]
    Was = jnp.zeros((D, D), f32).at[:, 0].set(va_s)
    Wat = jnp.zeros((D, D), f32).at[:, 0].set(va_t)
    cas = jnp.zeros((1, D), f32).at[0, 0].set(ca_s)
    cat = jnp.zeros((1, D), f32).at[0, 0].set(ca_t)

    pad_n = ((0, NP - N_NODES), (0, 0))
    sfp = jnp.pad(source_feat, pad_n)
    tfp = jnp.pad(target_feat, pad_n)

    head, thl, thr, As, At = _pre_pass(
        sfp, tfp, W_nt, b_nt.reshape(1, D), W1, c1.reshape(1, D),
        Was, cas, Wat, cat)
    alpha_s = As[:, 0]
    alpha_t = At[:, 0]

    ei = edge_index.astype(jnp.int32)
    src3 = jnp.pad(ei[0], (0, EP - E), constant_values=N_NODES).reshape(
        NS * (CHUNKS // HCH), HCH, CK)
    tgt3 = jnp.pad(ei[1], (0, EP - E), constant_values=N_NODES).reshape(
        NS * (CHUNKS // HCH), HCH, CK)

    def _pack_half(th):
        y = th.reshape(NP, 2, 2, 16).transpose(0, 1, 3, 2)
        yb = y.astype(jnp.bfloat16).reshape(NP, DH // 2, 2)
        return lax.bitcast_convert_type(yb, jnp.float32)

    ul, ur, dd = _sc_aggregate(src3, tgt3, alpha_s, alpha_t,
                               _pack_half(thl), _pack_half(thr))

    out = _post_pass(
        ul, ur, dd.reshape(NP // 256, 1, 256),
        head, gat_bias.reshape(1, D), W_prep, b_prep.reshape(1, D),
        W_d0, b_d0.reshape(1, D), g_d0.reshape(1, D), be_d0.reshape(1, D),
        W_d1, b_d1.reshape(1, D), g_d1.reshape(1, D), be_d1.reshape(1, D),
        g_blk.reshape(1, D), be_blk.reshape(1, D),
        jnp.zeros((D, D), f32).at[:, 0].set(W_cls[:, 0]),
        jnp.zeros((1, D), f32).at[0, 0].set(b_cls[0]))

    return out[:N_NODES, 0:1]
